# Initial kernel scaffold; baseline (speedup 1.0000x reference)
import jax
import jax.numpy as jnp
from jax.experimental import pallas as pl


def kernel(x_app, x_loc, x_time, edge_index_app, edge_weight_app, edge_index_loc, edge_weight_loc, W1, b1, W2, b2, Wa, ba, target_node):
    raise NotImplementedError("write your pallas kernel here")


# speedup vs baseline: 4.6966x; 4.6966x over previous
import functools

import numpy as np
import jax
import jax.numpy as jnp
from jax import lax
from jax.experimental import pallas as pl
from jax.experimental.pallas import tpu as pltpu
from jax.experimental.pallas import tpu_sc as plsc

N = 10000
E = 320000
D = 128
H = 128
WALK_LEN = 5

NC = 2
NT = 16
L = 16
EPT = E // NT
UF = 10
NG = EPT // (L * UF)

_U32 = jnp.uint32
_I32 = jnp.int32
_F32 = jnp.float32


def _np_threefry2x32(k1, k2, x0, x1):
    x0 = np.uint32(x0)
    x1 = np.uint32(x1)
    ks0, ks1 = np.uint32(k1), np.uint32(k2)
    ks2 = np.uint32(ks0 ^ ks1 ^ np.uint32(0x1BD11BDA))
    rot = ((13, 15, 26, 6), (17, 29, 16, 24))

    def rotl(v, r):
        return np.uint32((np.uint32(v) << np.uint32(r)) | (np.uint32(v) >> np.uint32(32 - r)))

    x0 = np.uint32(x0 + ks0)
    x1 = np.uint32(x1 + ks1)
    ks = (ks0, ks1, ks2)
    for i in range(5):
        for r in rot[i % 2]:
            x0 = np.uint32(x0 + x1)
            x1 = rotl(x1, r)
            x1 = np.uint32(x0 ^ x1)
        x0 = np.uint32(x0 + ks[(i + 1) % 3])
        x1 = np.uint32(x1 + ks[(i + 2) % 3] + np.uint32(i + 1))
    return x0, x1


def _np_fold_in(key, d):
    return _np_threefry2x32(key[0], key[1], np.uint32(0), np.uint32(d))


def _step_keys():
    root = (np.uint32(0), np.uint32(42))
    out = np.zeros((2, WALK_LEN, 2), np.uint32)
    for g in range(2):
        gk = _np_fold_in(root, g)
        for i in range(WALK_LEN):
            sk = _np_fold_in(gk, i)
            out[g, i, 0] = sk[0]
            out[g, i, 1] = sk[1]
    return out


_STEP_KEYS = _step_keys()


def _tf_block(k1, k2, x0, x1):
    ks2 = k1 ^ k2 ^ np.uint32(0x1BD11BDA)
    ks = (k1, k2, ks2)
    rot = ((13, 15, 26, 6), (17, 29, 16, 24))
    x0 = x0 + k1
    x1 = x1 + k2
    for i in range(5):
        for r in rot[i % 2]:
            x0 = x0 + x1
            x1 = (x1 << np.uint32(r)) | (x1 >> np.uint32(32 - r))
            x1 = x0 ^ x1
        x0 = x0 + ks[(i + 1) % 3]
        x1 = x1 + ks[(i + 2) % 3] + np.uint32(i + 1)
    return x0, x1


def _vtake(vec, idx):
    return lax.gather(
        vec, idx[:, None],
        dimension_numbers=lax.GatherDimensionNumbers(
            offset_dims=(), collapsed_slice_dims=(0,), start_index_map=(0,)),
        slice_sizes=(1,),
        mode=lax.GatherScatterMode.PROMISE_IN_BOUNDS)


def _any_scalar(m):
    return plsc.all_reduce_population_count(m)[0] > 0


def _vmax_splat(x):
    iota = lax.iota(_I32, L)
    for sh in (8, 4, 2, 1):
        x = jnp.maximum(x, _vtake(x, iota ^ sh))
    return x


_LN2_HI = np.float32(0.69313812256)
_LN2_LO = np.float32(9.0580006145e-06)
_SQRT2 = np.float32(1.4142135624)
_TINY = np.float32(np.finfo(np.float32).tiny)


def _vlog(x):
    bits = plsc.bitcast(x, _I32)
    e = (bits >> 23) - 127
    m = plsc.bitcast((bits & 0x7FFFFF) | 0x3F800000, _F32)
    big = m > _SQRT2
    m = jnp.where(big, m * np.float32(0.5), m)
    e = jnp.where(big, e + 1, e)
    ef = e.astype(_F32)
    s = (m - np.float32(1.0)) / (m + np.float32(1.0))
    z = s * s
    p = z * (np.float32(2 / 3) + z * (np.float32(2 / 5) + z * (
        np.float32(2 / 7) + z * (np.float32(2 / 9) + z * (
            np.float32(2 / 11) + z * np.float32(2 / 13))))))
    lm = np.float32(2.0) * s + s * p
    return ef * _LN2_HI + (lm + ef * _LN2_LO)


def _gumbel_val(k1, k2, evec_i32, wvec):
    e_u = evec_i32.astype(_U32)
    o0, o1 = _tf_block(k1, k2, jnp.zeros((L,), _U32), e_u)
    bitsv = o0 ^ o1
    fb = (bitsv >> np.uint32(9)) | np.uint32(0x3F800000)
    u = plsc.bitcast(fb.astype(_U32), _F32) - np.float32(1.0)
    u = u + _TINY
    t = -_vlog(u)
    return _vlog(wvec / t)


def _walk_body(ei_app, ew_app, ei_loc, ew_loc, x_app, x_loc, x_time,
               keys_hbm, tgt_hbm,
               agg_app_o, tgt_app_o, agg_loc_o, tgt_loc_o, ftime_o,
               src_v, dst_v, w_v, keys_v, tgt_v, pub_v, cand_l, rows_v,
               tidx_v, sums_v, cand_sh, sem):
    c = lax.axis_index("c")
    s = lax.axis_index("s")
    base = s * EPT

    @pl.when(c == 0)
    def _():
        pltpu.sync_copy(ei_app.at[pl.ds(base, EPT)], src_v)
        pltpu.sync_copy(ei_app.at[pl.ds(E + base, EPT)], dst_v)
        pltpu.sync_copy(ew_app.at[pl.ds(base, EPT)], w_v)
        pltpu.sync_copy(keys_hbm.at[pl.ds(0, WALK_LEN * 2 * L)], keys_v)

    @pl.when(c == 1)
    def _():
        pltpu.sync_copy(ei_loc.at[pl.ds(base, EPT)], src_v)
        pltpu.sync_copy(ei_loc.at[pl.ds(E + base, EPT)], dst_v)
        pltpu.sync_copy(ew_loc.at[pl.ds(base, EPT)], w_v)
        pltpu.sync_copy(keys_hbm.at[pl.ds(WALK_LEN * 2 * L, WALK_LEN * 2 * L)],
                        keys_v)

    pltpu.sync_copy(tgt_hbm, tgt_v)

    iota = lax.iota(_I32, L)
    neg_inf = jnp.full((L,), -jnp.inf, _F32)
    zeros_i = jnp.zeros((L,), _I32)

    cur0 = tgt_v[...]
    nodes0 = jnp.where(iota == 0, cur0, 0)

    def step(i, carry):
        cur, nodes = carry
        k1 = plsc.bitcast(keys_v[pl.ds(i * 2 * L, L)], _U32)
        k2 = plsc.bitcast(keys_v[pl.ds(i * 2 * L + L, L)], _U32)

        def group(gi, c2):
            off = gi * (UF * L)
            masks = [src_v[pl.ds(off + j * L, L)] == cur for j in range(UF)]
            anym = functools.reduce(jnp.logical_or, masks)

            def slow(c3):
                def chunk(j, c4):
                    bv4, be4 = c4
                    coff = off + j * L
                    m = src_v[pl.ds(coff, L)] == cur

                    def hit(c5):
                        bv5, be5 = c5
                        lidx = coff + iota
                        val = _gumbel_val(k1, k2, base + lidx,
                                          w_v[pl.ds(coff, L)])
                        val = jnp.where(m, val, neg_inf)
                        bett = val > bv5
                        return (jnp.where(bett, val, bv5),
                                jnp.where(bett, lidx, be5))

                    return lax.cond(_any_scalar(m), hit, lambda c5: c5, c4)

                return lax.fori_loop(0, UF, chunk, c3)

            return lax.cond(_any_scalar(anym), slow, lambda c3: c3, c2)

        bv, be = lax.fori_loop(0, NG, group, (neg_inf, zeros_i))

        cmax = _vmax_splat(bv)
        lane = jnp.broadcast_to(plsc.all_reduce_ffs(bv == cmax), (L,))
        el = _vtake(be, lane)
        bd = plsc.load_gather(dst_v, [el])

        vbits = plsc.bitcast(cmax, _I32)
        pub = jnp.where(iota == 0, vbits, jnp.where(iota == 1, bd, 0))
        pub_v[...] = pub
        pltpu.sync_copy(pub_v, cand_sh.at[s])
        plsc.subcore_barrier()
        pltpu.sync_copy(cand_sh, cand_l)
        vals = plsc.bitcast(plsc.load_gather(cand_l, [iota, zeros_i]), _F32)
        dsts = plsc.load_gather(cand_l, [iota, zeros_i + 1])
        gmax = _vmax_splat(vals)
        lane2 = jnp.broadcast_to(plsc.all_reduce_ffs(vals == gmax), (L,))
        cur2 = _vtake(dsts, lane2)
        nodes2 = jnp.where(iota == i + 1, cur2, nodes)
        plsc.subcore_barrier()
        return cur2, nodes2

    _, nodes = lax.fori_loop(0, WALK_LEN, step, (cur0, nodes0))

    tidx_v[...] = jnp.where(iota < WALK_LEN + 1, nodes, 0)

    @pl.when(s == 0)
    def _():
        @pl.when(c == 0)
        def _():
            pltpu.async_copy(x_app.at[tidx_v], rows_v, sem).wait()
            for cc in range(D // L):
                sl = pl.ds(cc * L, L)
                acc = rows_v[0, sl]
                for r in range(1, WALK_LEN + 1):
                    acc = acc + rows_v[r, sl]
                sums_v[sl] = acc
            pltpu.sync_copy(sums_v, agg_app_o)
            pltpu.sync_copy(rows_v.at[0], tgt_app_o)
            pltpu.async_copy(x_time.at[tidx_v], rows_v, sem).wait()
            pltpu.sync_copy(rows_v.at[0], ftime_o)

        @pl.when(c == 1)
        def _():
            pltpu.async_copy(x_loc.at[tidx_v], rows_v, sem).wait()
            for cc in range(D // L):
                sl = pl.ds(cc * L, L)
                acc = rows_v[0, sl]
                for r in range(1, WALK_LEN + 1):
                    acc = acc + rows_v[r, sl]
                sums_v[sl] = acc
            pltpu.sync_copy(sums_v, agg_loc_o)
            pltpu.sync_copy(rows_v.at[0], tgt_loc_o)


_walk_call = pl.kernel(
    _walk_body,
    out_type=[jax.ShapeDtypeStruct((D,), _F32)] * 5,
    mesh=plsc.VectorSubcoreMesh(core_axis_name="c", subcore_axis_name="s",
                                num_cores=NC, num_subcores=NT),
    compiler_params=pltpu.CompilerParams(needs_layout_passes=False),
    scratch_types=[
        pltpu.VMEM((EPT,), _I32),
        pltpu.VMEM((EPT,), _I32),
        pltpu.VMEM((EPT,), _F32),
        pltpu.VMEM((WALK_LEN * 2 * L,), _I32),
        pltpu.VMEM((L,), _I32),
        pltpu.VMEM((L,), _I32),
        pltpu.VMEM((NT, L), _I32),
        pltpu.VMEM((L, D), _F32),
        pltpu.VMEM((L,), _I32),
        pltpu.VMEM((D,), _F32),
        pltpu.VMEM_SHARED((NT, L), _I32),
        pltpu.SemaphoreType.DMA,
    ],
)


def _tail_body(ta, aa, tl, al, ft, w1, b1, w2, b2, wa3, ba, out):
    f_app = jnp.maximum(
        jnp.dot(jnp.concatenate([ta[...], aa[...]], axis=1), w1[...],
                preferred_element_type=_F32) + b1[...], 0.0)
    f_loc = jnp.maximum(
        jnp.dot(jnp.concatenate([tl[...], al[...]], axis=1), w2[...],
                preferred_element_type=_F32) + b2[...], 0.0)
    f_time = ft[...]
    wa = wa3[...]

    def dotv(f, j):
        return jnp.sum(f[0, :] * wa[j, :])

    bias = ba[0, 0]
    s0 = dotv(f_app, 0) + dotv(f_loc, 1) + dotv(f_time, 2) + bias
    s1 = dotv(f_loc, 0) + dotv(f_time, 1) + dotv(f_app, 2) + bias
    s2 = dotv(f_time, 0) + dotv(f_app, 1) + dotv(f_loc, 2) + bias
    m = jnp.maximum(jnp.maximum(s0, s1), s2)
    e0 = jnp.exp(s0 - m)
    e1 = jnp.exp(s1 - m)
    e2 = jnp.exp(s2 - m)
    tot = e0 + e1 + e2
    out[...] = (e0 * f_app + e1 * f_loc + e2 * f_time) / tot


_tail_call = pl.pallas_call(
    _tail_body,
    out_shape=jax.ShapeDtypeStruct((1, D), _F32),
)


def kernel(x_app, x_loc, x_time, edge_index_app, edge_weight_app,
           edge_index_loc, edge_weight_loc, W1, b1, W2, b2, Wa, ba,
           target_node):
    keys = np.zeros((2, WALK_LEN * 2 * L), np.int32)
    for g in range(2):
        for i in range(WALK_LEN):
            keys[g, i * 2 * L:i * 2 * L + L] = np.int32(_STEP_KEYS[g, i, 0].view(np.int32))
            keys[g, i * 2 * L + L:(i + 1) * 2 * L] = np.int32(_STEP_KEYS[g, i, 1].view(np.int32))
    keys = jnp.asarray(keys.reshape(-1))
    tgt = jnp.full((L,), target_node, _I32)

    agg_app, tgt_app, agg_loc, tgt_loc, f_time = _walk_call(
        edge_index_app.astype(_I32).reshape(2 * E), edge_weight_app,
        edge_index_loc.astype(_I32).reshape(2 * E), edge_weight_loc,
        x_app, x_loc, x_time, keys, tgt)

    out = _tail_call(
        tgt_app.reshape(1, D), agg_app.reshape(1, D),
        tgt_loc.reshape(1, D), agg_loc.reshape(1, D),
        f_time.reshape(1, D),
        W1, b1.reshape(1, H), W2, b2.reshape(1, H),
        Wa.reshape(3, H), ba.reshape(1, 1))
    return out.reshape(H)
